# Initial kernel scaffold; baseline (speedup 1.0000x reference)
import jax
import jax.numpy as jnp
from jax.experimental import pallas as pl


def kernel(x, edge_index, W1_l, b1_l, W1_r, W2_l, b2_l, W2_r):
    raise NotImplementedError("write your pallas kernel here")


# speedup vs baseline: 6.1148x; 6.1148x over previous
import functools

import jax
import jax.numpy as jnp
from jax import lax
from jax.experimental import pallas as pl
from jax.experimental.pallas import tpu as pltpu
from jax.experimental.pallas import tpu_sc as plsc

N_NODES = 10000
D = 128
COLS = D // 2
N_EDGES = 320000

NC = 2
NS = 16
CHUNK = 128
EDGES_PER_TILE = N_EDGES // NS
CHUNKS_PER_TILE = -(-EDGES_PER_TILE // CHUNK)
PAD_EPT = CHUNKS_PER_TILE * CHUNK
ACC_ROWS = 10240
ROWS_PER_TILE = ACC_ROWS // NS
OUT_ROWS_LAST = N_NODES - (NS - 1) * ROWS_PER_TILE
CNT_W = 16
DUMMY = N_NODES
ZROWS = 128
TC_BLK = 1000

_MESH = plsc.VectorSubcoreMesh(core_axis_name="c", subcore_axis_name="s")
_CP = pltpu.CompilerParams(use_tc_tiling_on_sc=False)


@functools.partial(
    pl.kernel,
    out_type=jax.ShapeDtypeStruct((NC, N_NODES, COLS), jnp.float32),
    mesh=_MESH,
    scratch_types=[
        pltpu.VMEM((CHUNKS_PER_TILE, CHUNK), jnp.int32),
        pltpu.VMEM((1, CHUNK), jnp.int32),
        pltpu.VMEM((1, CHUNK), jnp.int32),
        pltpu.VMEM((CHUNK, COLS), jnp.float32),
        pltpu.VMEM((ZROWS, COLS), jnp.float32),
        pltpu.VMEM_SHARED((ACC_ROWS, COLS), jnp.float32),
        pltpu.SemaphoreType.DMA,
    ],
    compiler_params=_CP,
)
def _sc_feat(xc_hbm, packp_hbm, acc_out,
             packv, srcrow, dstrow, rows, zblk, accsh, sem_g):
    c = lax.axis_index("c")
    s = lax.axis_index("s")
    base = s * ROWS_PER_TILE

    @pl.loop(0, ZROWS)
    def _(i):
        @pl.loop(0, COLS, step=16)
        def _(j):
            zblk[i, pl.ds(j, 16)] = jnp.zeros((16,), jnp.float32)

    @pl.loop(0, ROWS_PER_TILE, step=ZROWS)
    def _(r):
        pltpu.sync_copy(zblk, accsh.at[pl.ds(base + r, ZROWS)])

    pltpu.sync_copy(packp_hbm.at[s], packv)
    plsc.subcore_barrier()

    @pl.loop(0, CHUNKS_PER_TILE)
    def _(j):
        @pl.loop(0, CHUNK, step=16)
        def _(k):
            v = packv[j, pl.ds(k, 16)]
            srcrow[0, pl.ds(k, 16)] = lax.bitwise_and(v, 0xFFFF)
            dstrow[0, pl.ds(k, 16)] = lax.shift_right_logical(v, 16)

        pltpu.async_copy(xc_hbm.at[c].at[srcrow.at[0]], rows, sem_g).wait()
        pltpu.sync_copy(rows, accsh.at[dstrow.at[0]], add=True)

    plsc.subcore_barrier()

    @pl.when(s < NS - 1)
    def _():
        pltpu.sync_copy(accsh.at[pl.ds(base, ROWS_PER_TILE)],
                        acc_out.at[c, pl.ds(base, ROWS_PER_TILE)])

    @pl.when(s == NS - 1)
    def _():
        pltpu.sync_copy(accsh.at[pl.ds(base, OUT_ROWS_LAST)],
                        acc_out.at[c, pl.ds(base, OUT_ROWS_LAST)])


@functools.partial(
    pl.kernel,
    out_type=jax.ShapeDtypeStruct((NC, N_NODES, CNT_W), jnp.float32),
    mesh=_MESH,
    scratch_types=[
        pltpu.VMEM((CHUNKS_PER_TILE, CHUNK), jnp.int32),
        pltpu.VMEM((1, CHUNK), jnp.int32),
        pltpu.VMEM((CHUNK, CNT_W), jnp.float32),
        pltpu.VMEM((ZROWS, CNT_W), jnp.float32),
        pltpu.VMEM_SHARED((ACC_ROWS, CNT_W), jnp.float32),
    ],
    compiler_params=_CP,
)
def _sc_count(packp_hbm, cnt_out, packv, dstrow, onesv, zblk, cntsh):
    c = lax.axis_index("c")
    s = lax.axis_index("s")
    base = s * ROWS_PER_TILE

    @pl.loop(0, ZROWS)
    def _(i):
        zblk[i, :] = jnp.zeros((CNT_W,), jnp.float32)

    @pl.loop(0, CHUNK)
    def _(i):
        onesv[i, :] = jnp.ones((CNT_W,), jnp.float32)

    @pl.loop(0, ROWS_PER_TILE, step=ZROWS)
    def _(r):
        pltpu.sync_copy(zblk, cntsh.at[pl.ds(base + r, ZROWS)])

    pltpu.sync_copy(packp_hbm.at[s], packv)
    plsc.subcore_barrier()

    @pl.loop(0, CHUNKS_PER_TILE)
    def _(j):
        @pl.loop(0, CHUNK, step=16)
        def _(k):
            dstrow[0, pl.ds(k, 16)] = lax.shift_right_logical(
                packv[j, pl.ds(k, 16)], 16)

        pltpu.sync_copy(onesv, cntsh.at[dstrow.at[0]], add=True)

    plsc.subcore_barrier()

    @pl.when(s < NS - 1)
    def _():
        pltpu.sync_copy(cntsh.at[pl.ds(base, ROWS_PER_TILE)],
                        cnt_out.at[c, pl.ds(base, ROWS_PER_TILE)])

    @pl.when(s == NS - 1)
    def _():
        pltpu.sync_copy(cntsh.at[pl.ds(base, OUT_ROWS_LAST)],
                        cnt_out.at[c, pl.ds(base, OUT_ROWS_LAST)])


def _tc_body(relu, split_out, acc_ref, cnt_ref, x_ref, wl_ref, b_ref,
             wr_ref, o_ref):
    agg = jnp.concatenate([acc_ref[0], acc_ref[1]], axis=1)
    xv = jnp.concatenate([x_ref[0], x_ref[1]], axis=1)
    csum = (cnt_ref[0, :, 0:1] + cnt_ref[1, :, 0:1]) * 0.5
    agg = agg / jnp.maximum(csum, 1.0)
    y = jnp.dot(agg, wl_ref[...], preferred_element_type=jnp.float32)
    y = y + b_ref[...] + jnp.dot(xv, wr_ref[...],
                                 preferred_element_type=jnp.float32)
    if relu:
        y = jnp.maximum(y, 0.0)
    if split_out:
        o_ref[0] = y[:, :COLS]
        o_ref[1] = y[:, COLS:]
    else:
        o_ref[...] = y


def _tc_combine(acc, cnt, xc, wl_t, b2d, wr_t, relu, split_out):
    if split_out:
        out_shape = jax.ShapeDtypeStruct((NC, N_NODES, COLS), jnp.float32)
        out_spec = pl.BlockSpec((NC, TC_BLK, COLS), lambda i: (0, i, 0))
    else:
        out_shape = jax.ShapeDtypeStruct((N_NODES, D), jnp.float32)
        out_spec = pl.BlockSpec((TC_BLK, D), lambda i: (i, 0))
    return pl.pallas_call(
        functools.partial(_tc_body, relu, split_out),
        grid=(N_NODES // TC_BLK,),
        in_specs=[
            pl.BlockSpec((NC, TC_BLK, COLS), lambda i: (0, i, 0)),
            pl.BlockSpec((NC, TC_BLK, CNT_W), lambda i: (0, i, 0)),
            pl.BlockSpec((NC, TC_BLK, COLS), lambda i: (0, i, 0)),
            pl.BlockSpec((D, D), lambda i: (0, 0)),
            pl.BlockSpec((1, D), lambda i: (0, 0)),
            pl.BlockSpec((D, D), lambda i: (0, 0)),
        ],
        out_specs=out_spec,
        out_shape=out_shape,
    )(acc, cnt, xc, wl_t, b2d, wr_t)


@jax.jit
def kernel(x, edge_index, W1_l, b1_l, W1_r, W2_l, b2_l, W2_r):
    src = edge_index[0].astype(jnp.int32)
    dst = edge_index[1].astype(jnp.int32)
    packed = jnp.bitwise_or(jnp.left_shift(dst, 16), src)
    packed = packed.reshape(NS, EDGES_PER_TILE)
    pad = PAD_EPT - EDGES_PER_TILE
    packp = jnp.pad(packed, ((0, 0), (0, pad)),
                    constant_values=DUMMY << 16
                    ).reshape(NS, CHUNKS_PER_TILE, CHUNK)

    xc = jnp.stack([x[:, :COLS], x[:, COLS:]], axis=0)

    cnt = _sc_count(packp)
    acc1 = _sc_feat(xc, packp)
    hc = _tc_combine(acc1, cnt, xc, W1_l.T, b1_l.reshape(1, D), W1_r.T,
                     relu=True, split_out=True)
    acc2 = _sc_feat(hc, packp)
    return _tc_combine(acc2, cnt, hc, W2_l.T, b2_l.reshape(1, D), W2_r.T,
                       relu=False, split_out=False)
